# baseline (speedup 1.0000x reference)
import functools

import jax
import jax.numpy as jnp
from jax import lax
from jax.experimental import pallas as pl
from jax.experimental.pallas import tpu as pltpu
from jax.experimental.pallas import tpu_sc as plsc

NUM_CLASSES = 37
NUM_INSTANCE = 64
FEAT = 256
N_ROWS = 65536
LANES = 16
FEAT_V = FEAT // LANES

NUM_CORES = 2
NUM_SUBCORES = 16
NW = NUM_CORES * NUM_SUBCORES
ROWS_PER_W = N_ROWS // NW
CHUNK = 128
NCHUNK = ROWS_PER_W // CHUNK
GROUPS = CHUNK // LANES
CPAD = 40
SUMS_LEN = CPAD * FEAT
CNTS_LEN = CPAD * LANES


def _sc_body(feat_hbm, lab_hbm, sums_out, cnts_out,
             rows_v, rows_v2, labels_v, sums_flat, cnts_flat, sem):
    cid = lax.axis_index("c")
    sid = lax.axis_index("s")
    wid = cid * NUM_SUBCORES + sid

    zeros16 = jnp.zeros((LANES,), jnp.float32)
    ones16 = jnp.ones((LANES,), jnp.float32)
    lane_iota = lax.iota(jnp.int32, LANES)

    def _zero(ref, n):
        def body(i, carry):
            ref[pl.ds(i * LANES, LANES)] = zeros16
            return carry
        lax.fori_loop(0, n // LANES, body, 0)

    _zero(sums_flat, SUMS_LEN)
    _zero(cnts_flat, CNTS_LEN)

    pltpu.sync_copy(lab_hbm.at[wid], labels_v)

    base = wid * ROWS_PER_W

    def _chunk_src(k):
        return feat_hbm.at[pl.ds(base + k * CHUNK, CHUNK), :]

    def cnt_body(k, carry):
        for g in range(GROUPS):
            lv = labels_v[k, pl.ds(g * LANES, LANES)]
            plsc.addupdate_scatter(cnts_flat, [lv * LANES + lane_iota], ones16)
        return carry
    lax.fori_loop(0, NCHUNK, cnt_body, 0)

    def _accum_chunk(k, buf):
        def g_body(g, carry):
            lv = labels_v[k, pl.ds(g * LANES, LANES)]
            for m in range(LANES):
                lab_b = lv.at[jnp.full((LANES,), m, jnp.int32)].get(
                    mode="promise_in_bounds")
                rowbase = lab_b * FEAT + lane_iota
                for j in range(FEAT_V):
                    vals = buf[g * LANES + m, pl.ds(j * LANES, LANES)]
                    plsc.addupdate_scatter(
                        sums_flat, [rowbase + (j * LANES)], vals)
            return carry
        lax.fori_loop(0, GROUPS, g_body, 0)

    pltpu.async_copy(_chunk_src(0), rows_v, sem)

    def pair_body(p, carry):
        k0 = 2 * p
        pltpu.make_async_copy(_chunk_src(k0), rows_v, sem).wait()
        pltpu.async_copy(_chunk_src(k0 + 1), rows_v2, sem)
        _accum_chunk(k0, rows_v)
        pltpu.make_async_copy(_chunk_src(k0 + 1), rows_v2, sem).wait()

        @pl.when(k0 + 2 < NCHUNK)
        def _():
            pltpu.async_copy(_chunk_src(k0 + 2), rows_v, sem)

        _accum_chunk(k0 + 1, rows_v2)
        return carry

    lax.fori_loop(0, NCHUNK // 2, pair_body, 0)

    pltpu.sync_copy(sums_flat, sums_out.at[wid])
    pltpu.sync_copy(cnts_flat, cnts_out.at[wid])


_sc_accum = functools.partial(
    pl.kernel,
    out_type=(
        jax.ShapeDtypeStruct((NW, SUMS_LEN), jnp.float32),
        jax.ShapeDtypeStruct((NW, CNTS_LEN), jnp.float32),
    ),
    mesh=plsc.VectorSubcoreMesh(core_axis_name="c", subcore_axis_name="s"),
    scratch_types=[
        pltpu.VMEM((CHUNK, FEAT), jnp.float32),
        pltpu.VMEM((CHUNK, FEAT), jnp.float32),
        pltpu.VMEM((NCHUNK, CHUNK), jnp.int32),
        pltpu.VMEM((SUMS_LEN,), jnp.float32),
        pltpu.VMEM((CNTS_LEN,), jnp.float32),
        pltpu.SemaphoreType.DMA,
    ],
    compiler_params=pltpu.CompilerParams(needs_layout_passes=False),
)(_sc_body)


def _combine_body(p_ref, c_ref, q_ref, t_ref, o_ref):
    sums = p_ref[0, :NUM_CLASSES]
    cnts = c_ref[0, :NUM_CLASSES]
    for w in range(1, NW):
        sums = sums + p_ref[w, :NUM_CLASSES]
        cnts = cnts + c_ref[w, :NUM_CLASSES]
    cnt = jnp.sum(cnts, axis=1, keepdims=True)
    mean = sums / jnp.maximum(cnt, 1.0)
    present = cnt > 0.0
    tail = t_ref[...]
    for j in range(NUM_INSTANCE):
        hit = (tail == j) & present
        o_ref[:, j, :] = jnp.where(hit, mean, q_ref[:, j, :])


def _combine(sums, cnts, queue, tail2d):
    return pl.pallas_call(
        _combine_body,
        out_shape=jax.ShapeDtypeStruct((NUM_CLASSES, NUM_INSTANCE, FEAT),
                                       jnp.float32),
    )(sums, cnts, queue, tail2d)


def kernel(features, pid_labels, large_batch_queue, tail):
    labels_r = pid_labels.reshape(NW, NCHUNK, CHUNK)
    sums, cnts = _sc_accum(features, labels_r)
    sums = sums.reshape(NW, CPAD, FEAT)
    cnts = cnts.reshape(NW, CPAD, LANES)
    return _combine(sums, cnts, large_batch_queue,
                    tail.reshape(NUM_CLASSES, 1))


# speedup vs baseline: 1.4380x; 1.4380x over previous
import functools

import jax
import jax.numpy as jnp
from jax import lax
from jax.experimental import pallas as pl
from jax.experimental.pallas import tpu as pltpu
from jax.experimental.pallas import tpu_sc as plsc

NUM_CLASSES = 37
NUM_INSTANCE = 64
FEAT = 256
N_ROWS = 65536
LANES = 16
HFEAT = 128
NH = N_ROWS * FEAT // HFEAT

NUM_CORES = 2
NUM_SUBCORES = 16
NW = NUM_CORES * NUM_SUBCORES
HROWS_PER_W = NH // NW
CHUNK = 128
NCHUNK = HROWS_PER_W // CHUNK
LROWS_PER_W = N_ROWS // NW
LCHUNK = 128
NLCHUNK = LROWS_PER_W // LCHUNK
CPAD = 40
APAD = 2 * CPAD


def _sc_body(feat_hbm, idx_hbm, lab_hbm, sums_out, cnts_out,
             rows_v, rows_v2, idx_v, labels_v, ones_v, sums_acc, cnts_acc,
             sem):
    cid = lax.axis_index("c")
    sid = lax.axis_index("s")
    wid = cid * NUM_SUBCORES + sid

    zeros16 = jnp.zeros((LANES,), jnp.float32)
    ones16 = jnp.ones((LANES,), jnp.float32)

    @pl.when(sid == 0)
    def _():
        for i in range(APAD):
            for j in range(HFEAT // LANES):
                rows_v[i, pl.ds(j * LANES, LANES)] = zeros16
        for i in range(CPAD):
            ones_v[i, :] = zeros16
        pltpu.sync_copy(rows_v.at[pl.ds(0, APAD), :], sums_acc)
        pltpu.sync_copy(ones_v.at[pl.ds(0, CPAD), :], cnts_acc)

    for i in range(LCHUNK):
        ones_v[i, :] = ones16

    pltpu.sync_copy(idx_hbm.at[wid], idx_v)
    pltpu.sync_copy(lab_hbm.at[wid], labels_v)
    plsc.subcore_barrier()

    for k in range(NLCHUNK):
        pltpu.sync_copy(ones_v, cnts_acc.at[labels_v.at[k]], add=True)

    base = wid * HROWS_PER_W

    def _chunk_src(k):
        return feat_hbm.at[pl.ds(base + k * CHUNK, CHUNK), :]

    pltpu.async_copy(_chunk_src(0), rows_v, sem)

    def pair_body(p, carry):
        k0 = 2 * p
        pltpu.make_async_copy(_chunk_src(k0), rows_v, sem).wait()
        pltpu.async_copy(_chunk_src(k0 + 1), rows_v2, sem)
        pltpu.sync_copy(rows_v, sums_acc.at[idx_v.at[k0]], add=True)
        pltpu.make_async_copy(_chunk_src(k0 + 1), rows_v2, sem).wait()

        @pl.when(k0 + 2 < NCHUNK)
        def _():
            pltpu.async_copy(_chunk_src(k0 + 2), rows_v, sem)

        pltpu.sync_copy(rows_v2, sums_acc.at[idx_v.at[k0 + 1]], add=True)
        return carry

    lax.fori_loop(0, NCHUNK // 2, pair_body, 0)

    plsc.subcore_barrier()

    @pl.when(sid == 0)
    def _():
        pltpu.sync_copy(sums_acc, sums_out.at[cid])
        pltpu.sync_copy(cnts_acc, cnts_out.at[cid])


_sc_accum = functools.partial(
    pl.kernel,
    out_type=(
        jax.ShapeDtypeStruct((NUM_CORES, APAD, HFEAT), jnp.float32),
        jax.ShapeDtypeStruct((NUM_CORES, CPAD, LANES), jnp.float32),
    ),
    mesh=plsc.VectorSubcoreMesh(core_axis_name="c", subcore_axis_name="s"),
    scratch_types=[
        pltpu.VMEM((CHUNK, HFEAT), jnp.float32),
        pltpu.VMEM((CHUNK, HFEAT), jnp.float32),
        pltpu.VMEM((NCHUNK, CHUNK), jnp.int32),
        pltpu.VMEM((NLCHUNK, LCHUNK), jnp.int32),
        pltpu.VMEM((LCHUNK, LANES), jnp.float32),
        pltpu.VMEM_SHARED((APAD, HFEAT), jnp.float32),
        pltpu.VMEM_SHARED((CPAD, LANES), jnp.float32),
        pltpu.SemaphoreType.DMA,
    ],
    compiler_params=pltpu.CompilerParams(use_tc_tiling_on_sc=False),
)(_sc_body)


def _combine_body(p_ref, c_ref, q_ref, t_ref, o_ref):
    sums = p_ref[0, :NUM_CLASSES]
    cnts = c_ref[0, :NUM_CLASSES]
    for w in range(1, NUM_CORES):
        sums = sums + p_ref[w, :NUM_CLASSES]
        cnts = cnts + c_ref[w, :NUM_CLASSES]
    cnt = cnts[:, 0:1]
    mean = sums / jnp.maximum(cnt, 1.0)
    present = cnt > 0.0
    tail = t_ref[...]
    for j in range(NUM_INSTANCE):
        hit = (tail == j) & present
        o_ref[:, j, :] = jnp.where(hit, mean, q_ref[:, j, :])


def _combine(sums, cnts, queue, tail2d):
    return pl.pallas_call(
        _combine_body,
        out_shape=jax.ShapeDtypeStruct((NUM_CLASSES, NUM_INSTANCE, FEAT),
                                       jnp.float32),
    )(sums, cnts, queue, tail2d)


def kernel(features, pid_labels, large_batch_queue, tail):
    feat_h = features.reshape(N_ROWS // 8, 8, 2, HFEAT)
    feat_h = feat_h.transpose(0, 2, 1, 3).reshape(NH, HFEAT)
    ct = jnp.arange(2, dtype=jnp.int32)
    idx2 = (pid_labels.reshape(N_ROWS // 8, 1, 8) * 2 + ct[None, :, None])
    idx2 = idx2.reshape(NW, NCHUNK, CHUNK)
    labels_r = pid_labels.reshape(NW, NLCHUNK, LCHUNK)
    sums, cnts = _sc_accum(feat_h, idx2, labels_r)
    sums = sums.reshape(NUM_CORES, CPAD, FEAT)
    return _combine(sums, cnts, large_batch_queue,
                    tail.reshape(NUM_CLASSES, 1))
